# baseline (speedup 1.0000x reference)
import jax
import jax.numpy as jnp
from jax import lax
from jax.experimental import pallas as pl
from jax.experimental.pallas import tpu as pltpu
from jax.experimental.pallas import tpu_sc as plsc

SEQ_LEN = 384
REAL_LEN = 355
CLS_ID = 101
SEP_ID = 102

B = 1024
D = 128
ROWS = B * SEQ_LEN
NW = 32
ROWS_PER_W = ROWS // NW
CHUNK = 128
NCHUNK = ROWS_PER_W // CHUNK
CHUNKS_PER_SEQ = SEQ_LEN // CHUNK


NBUF = 4


def _sc_body(ids_hbm, word_hbm, bias_hbm, out_hbm, idx_v, bias_v, g_v,
             g_sem, o_sem):
    nc = 2
    wid = lax.axis_index("s") * nc + lax.axis_index("c")
    base_chunk = wid * NCHUNK

    pltpu.sync_copy(ids_hbm.at[pl.ds(base_chunk, NCHUNK), :], idx_v)
    pltpu.sync_copy(bias_hbm, bias_v)

    def gather_start(c, b):
        pltpu.async_copy(word_hbm.at[idx_v.at[c]], g_v.at[b], g_sem.at[b])

    def gather_wait(c, b):
        pltpu.make_async_copy(
            word_hbm.at[idx_v.at[c]], g_v.at[b], g_sem.at[b]
        ).wait()

    def out_start(c, b):
        pltpu.async_copy(
            g_v.at[b],
            out_hbm.at[pl.ds((base_chunk + c) * CHUNK, CHUNK), :],
            o_sem.at[b],
        )

    def out_wait(b):
        pltpu.make_async_copy(
            g_v.at[b], out_hbm.at[pl.ds(0, CHUNK), :], o_sem.at[b]
        ).wait()

    gather_start(0, 0)
    gather_start(1, 1)

    def iter_body(i, carry):
        for b in range(NBUF):
            c = i * NBUF + b
            gather_wait(c, b)
            bias_off = lax.rem(c, CHUNKS_PER_SEQ) * CHUNK

            def row_body(r, rc):
                br = bias_off + r
                for d in range(D // 16):
                    sl = pl.ds(d * 16, 16)
                    plsc.addupdate(g_v.at[b, r, sl], bias_v[br, sl])
                return rc

            lax.fori_loop(0, CHUNK, row_body, 0, unroll=4)
            out_start(c, b)
            b2 = (b + 2) % NBUF
            c2 = c + 2

            @pl.when(c2 < NCHUNK)
            def _():
                @pl.when(c >= 2)
                def _():
                    out_wait(b2)

                gather_start(c2, b2)
        return carry

    lax.fori_loop(0, NCHUNK // NBUF, iter_body, 0)
    for b in range(NBUF):
        out_wait(b)


@jax.jit
def _run(ids2d, word_emb, bias):
    mesh = plsc.VectorSubcoreMesh(core_axis_name="c", subcore_axis_name="s")
    kfn = pl.kernel(
        _sc_body,
        out_type=jax.ShapeDtypeStruct((ROWS, D), jnp.float32),
        mesh=mesh,
        scratch_types=[
            pltpu.VMEM((NCHUNK, CHUNK), jnp.int32),
            pltpu.VMEM((SEQ_LEN, D), jnp.float32),
            pltpu.VMEM((NBUF, CHUNK, D), jnp.float32),
            pltpu.SemaphoreType.DMA((NBUF,)),
            pltpu.SemaphoreType.DMA((NBUF,)),
        ],
    )
    return kfn(ids2d, word_emb, bias)


def kernel(paragraph_ids, question_ids, word_emb, type_emb, pos_emb):
    Bq, Lp = paragraph_ids.shape
    Lq = question_ids.shape[1]
    dt = paragraph_ids.dtype
    cls_col = jnp.full((Bq, 1), CLS_ID, dtype=dt)
    sep_col = jnp.full((Bq, 1), SEP_ID, dtype=dt)
    pad_blk = jnp.zeros((Bq, SEQ_LEN - REAL_LEN), dtype=dt)
    ids = jnp.concatenate(
        [cls_col, paragraph_ids, sep_col, question_ids, sep_col, pad_blk],
        axis=1,
    )
    ids2d = ids.reshape(ROWS // CHUNK, CHUNK)

    t = jnp.arange(SEQ_LEN)
    type_idx = ((t >= 1 + Lp + 1) & (t < REAL_LEN)).astype(jnp.int32)
    bias = pos_emb + jnp.take(type_emb, type_idx, axis=0)
    bias = jnp.where((t < REAL_LEN)[:, None], bias, -word_emb[0][None, :])

    out = _run(ids2d, word_emb, bias)
    return out.reshape(B, SEQ_LEN, D)


# speedup vs baseline: 1.0098x; 1.0098x over previous
import jax
import jax.numpy as jnp
from jax import lax
from jax.experimental import pallas as pl
from jax.experimental.pallas import tpu as pltpu
from jax.experimental.pallas import tpu_sc as plsc

SEQ_LEN = 384
REAL_LEN = 355
CLS_ID = 101
SEP_ID = 102

B = 1024
D = 128
ROWS = B * SEQ_LEN
NW = 32
ROWS_PER_W = ROWS // NW
CHUNK = 128
NCHUNK = ROWS_PER_W // CHUNK
CHUNKS_PER_SEQ = SEQ_LEN // CHUNK


NBUF = 4


def _sc_body(ids_hbm, word_hbm, bias_hbm, out_hbm, idx_v, bias_sh, g_v,
             g_sem, o_sem):
    nc = 2
    wid = lax.axis_index("s") * nc + lax.axis_index("c")
    base_chunk = wid * NCHUNK

    pltpu.sync_copy(ids_hbm.at[pl.ds(base_chunk, NCHUNK), :], idx_v)

    @pl.when(lax.axis_index("s") == 0)
    def _():
        pltpu.sync_copy(bias_hbm, bias_sh)

    plsc.subcore_barrier()

    def gather_start(c, b):
        pltpu.async_copy(
            word_hbm.at[idx_v.at[c]], g_v.at[b], g_sem.at[b], add=True
        )

    def gather_wait(c, b):
        pltpu.make_async_copy(
            word_hbm.at[idx_v.at[c]], g_v.at[b], g_sem.at[b]
        ).wait()

    def out_start(c, b):
        pltpu.async_copy(
            g_v.at[b],
            out_hbm.at[pl.ds((base_chunk + c) * CHUNK, CHUNK), :],
            o_sem.at[b],
        )

    def out_wait(b):
        pltpu.make_async_copy(
            g_v.at[b], out_hbm.at[pl.ds(0, CHUNK), :], o_sem.at[b]
        ).wait()

    def bias_init(c, b):
        off = lax.rem(c, CHUNKS_PER_SEQ) * CHUNK
        pltpu.sync_copy(bias_sh.at[pl.ds(off, CHUNK), :], g_v.at[b])

    bias_init(0, 0)
    gather_start(0, 0)
    bias_init(1, 1)
    gather_start(1, 1)

    def iter_body(i, carry):
        for b in range(NBUF):
            c = i * NBUF + b
            gather_wait(c, b)
            out_start(c, b)
            b2 = (b + 2) % NBUF
            c2 = c + 2

            @pl.when(c2 < NCHUNK)
            def _():
                @pl.when(c >= 2)
                def _():
                    out_wait(b2)

                bias_init(c2, b2)
                gather_start(c2, b2)
        return carry

    lax.fori_loop(0, NCHUNK // NBUF, iter_body, 0)
    for b in range(NBUF):
        out_wait(b)


@jax.jit
def _run(ids2d, word_emb, bias):
    mesh = plsc.VectorSubcoreMesh(core_axis_name="c", subcore_axis_name="s")
    kfn = pl.kernel(
        _sc_body,
        out_type=jax.ShapeDtypeStruct((ROWS, D), jnp.float32),
        mesh=mesh,
        scratch_types=[
            pltpu.VMEM((NCHUNK, CHUNK), jnp.int32),
            pltpu.VMEM_SHARED((SEQ_LEN, D), jnp.float32),
            pltpu.VMEM((NBUF, CHUNK, D), jnp.float32),
            pltpu.SemaphoreType.DMA((NBUF,)),
            pltpu.SemaphoreType.DMA((NBUF,)),
        ],
    )
    return kfn(ids2d, word_emb, bias)


def kernel(paragraph_ids, question_ids, word_emb, type_emb, pos_emb):
    Bq, Lp = paragraph_ids.shape
    Lq = question_ids.shape[1]
    dt = paragraph_ids.dtype
    cls_col = jnp.full((Bq, 1), CLS_ID, dtype=dt)
    sep_col = jnp.full((Bq, 1), SEP_ID, dtype=dt)
    pad_blk = jnp.zeros((Bq, SEQ_LEN - REAL_LEN), dtype=dt)
    ids = jnp.concatenate(
        [cls_col, paragraph_ids, sep_col, question_ids, sep_col, pad_blk],
        axis=1,
    )
    ids2d = ids.reshape(ROWS // CHUNK, CHUNK)

    t = jnp.arange(SEQ_LEN)
    type_idx = ((t >= 1 + Lp + 1) & (t < REAL_LEN)).astype(jnp.int32)
    bias = pos_emb + jnp.take(type_emb, type_idx, axis=0)
    bias = jnp.where((t < REAL_LEN)[:, None], bias, -word_emb[0][None, :])

    out = _run(ids2d, word_emb, bias)
    return out.reshape(B, SEQ_LEN, D)
